# baseline (speedup 1.0000x reference)
import functools

import jax
import jax.numpy as jnp
from jax import lax
from jax.experimental import pallas as pl
from jax.experimental.pallas import tpu as pltpu
from jax.experimental.pallas import tpu_sc as plsc

_VOCAB_N = 100000
_D = 300
_B = 1024
_CTX = 20

_NC = 2
_NS = 16
_NW = _NC * _NS

_ROWS = _B * _CTX
_ROWS_PER_W = _ROWS // _NW
_CH = 128
_NCHUNK = _ROWS_PER_W // _CH


_DP = 384


def _sc_gather(table, idx_flat):
    mesh = plsc.VectorSubcoreMesh(core_axis_name="c", subcore_axis_name="s")

    @functools.partial(
        pl.kernel,
        mesh=mesh,
        out_type=jax.ShapeDtypeStruct((_ROWS, _DP), jnp.float32),
        scratch_types=[
            pltpu.VMEM((_CH,), jnp.int32),
            pltpu.VMEM((_CH, 256), jnp.float32),
            pltpu.VMEM((_CH, 128), jnp.float32),
            pltpu.SemaphoreType.DMA,
        ],
    )
    def k(table_hbm, idx_hbm, rows_hbm, idx_v, r01, r2, sem):
        wid = lax.axis_index("s") * _NC + lax.axis_index("c")
        base = wid * _ROWS_PER_W

        def chunk(t, carry):
            off = base + t * _CH
            pltpu.sync_copy(idx_hbm.at[pl.ds(off, _CH)], idx_v)
            pltpu.async_copy(table_hbm.at[idx_v, pl.ds(0, 256)], r01, sem).wait()
            o2 = pl.multiple_of(t * 0 + 256, 128)
            pltpu.async_copy(table_hbm.at[idx_v, pl.ds(o2, 128)], r2, sem).wait()
            pltpu.sync_copy(r01, rows_hbm.at[pl.ds(off, _CH), pl.ds(0, 256)])
            pltpu.sync_copy(r2, rows_hbm.at[pl.ds(off, _CH), pl.ds(256, 128)])
            return carry

        lax.fori_loop(0, _NCHUNK, chunk, 0)

    return k(table, idx_flat)


def _pool_body(e_ref, o_ref):
    ep = e_ref[...]
    e = ep[..., :_D]
    sq = jnp.sum(e * e, axis=-1, keepdims=True)
    norm = jnp.sqrt(sq)
    scale = jnp.minimum(1.0, 1.0 / (norm + 1e-7))
    o_ref[...] = jnp.mean(e * scale, axis=1)


def _pool(rows):
    bb = 128
    e = rows.reshape(_B, _CTX, _DP)
    return pl.pallas_call(
        _pool_body,
        grid=(_B // bb,),
        in_specs=[pl.BlockSpec((bb, _CTX, _DP), lambda i: (i, 0, 0))],
        out_specs=pl.BlockSpec((bb, _D), lambda i: (i, 0)),
        out_shape=jax.ShapeDtypeStruct((_B, _D), jnp.float32),
    )(e)


_MB = 2048
_NFULL = _VOCAB_N // _MB
_EP_BASE = _NFULL * _MB
_EP_N = _VOCAB_N - _EP_BASE
_EP_PAD = 100096 - _EP_BASE


def _mm2_body(x_ref, b_ref, w_hbm, o_ref, xb_ref, wbuf, obuf, wsem, osem):
    xb_ref[...] = x_ref[...].astype(jnp.bfloat16)
    t0 = (b_ref[0, 0] * 0.0).astype(jnp.int32)

    def wcp(row_off, slot):
        return pltpu.make_async_copy(
            w_hbm.at[pl.ds(row_off, _MB)], wbuf.at[slot], wsem.at[slot]
        )

    def ocp(col_off, slot, size):
        off = pl.multiple_of(t0 + col_off, 128)
        return pltpu.make_async_copy(
            obuf.at[slot, slice(None), pl.ds(0, size)],
            o_ref.at[:, pl.ds(off, size)],
            osem.at[slot],
        )

    def compute(slot, col_off, nsize):
        wb = wbuf[slot, pl.ds(0, nsize), :].astype(jnp.bfloat16)
        acc = lax.dot_general(
            xb_ref[...], wb, (((1,), (1,)), ((), ())),
            preferred_element_type=jnp.float32,
        )
        boff = pl.multiple_of(t0 + col_off, 128)
        bv = b_ref[0, pl.ds(boff, nsize)]
        obuf[slot, slice(None), pl.ds(0, nsize)] = acc + bv[None, :]

    wcp(0, 0).start()
    wcp(_MB, 1).start()

    def step(j, carry):
        slot = lax.rem(j, 2)
        wcp(j * _MB, slot).wait()

        @pl.when(j >= 2)
        def _():
            ocp(j * _MB, slot, _MB).wait()

        compute(slot, j * _MB, _MB)
        ocp(j * _MB, slot, _MB).start()

        @pl.when(j + 2 < _NFULL)
        def _():
            wcp((j + 2) * _MB, slot).start()

        return carry

    lax.fori_loop(0, _NFULL, step, 0)

    ep_w = pltpu.make_async_copy(
        w_hbm.at[pl.ds(_EP_BASE, _EP_N)],
        wbuf.at[0, pl.ds(0, _EP_N)],
        wsem.at[0],
    )
    ep_w.start()
    ocp((_NFULL - 2) * _MB, 0, _MB).wait()
    ep_w.wait()
    compute(0, _EP_BASE, _EP_PAD)
    ocp(_EP_BASE, 0, _EP_PAD).start()
    ocp((_NFULL - 1) * _MB, 1, _MB).wait()
    ocp(_EP_BASE, 0, _EP_PAD).wait()


def _project(x, W, b):
    b2 = b.reshape(1, _VOCAB_N)
    return pl.pallas_call(
        _mm2_body,
        in_specs=[
            pl.BlockSpec((_B, _D), lambda: (0, 0)),
            pl.BlockSpec((1, _VOCAB_N), lambda: (0, 0)),
            pl.BlockSpec(memory_space=pl.ANY),
        ],
        out_specs=pl.BlockSpec(memory_space=pl.ANY),
        out_shape=jax.ShapeDtypeStruct((_B, _VOCAB_N), jnp.float32),
        scratch_shapes=[
            pltpu.VMEM((_B, _D), jnp.bfloat16),
            pltpu.VMEM((2, _MB, _D), jnp.float32),
            pltpu.VMEM((2, _B, _MB), jnp.float32),
            pltpu.SemaphoreType.DMA((2,)),
            pltpu.SemaphoreType.DMA((2,)),
        ],
    )(x, b2, W)


def kernel(inputs_, emb_table, W, b):
    idx_flat = inputs_.reshape(-1).astype(jnp.int32)
    rows = _sc_gather(emb_table, idx_flat)
    x = _pool(rows)
    return _project(x, W, b)


# speedup vs baseline: 1.0028x; 1.0028x over previous
import functools

import jax
import jax.numpy as jnp
from jax import lax
from jax.experimental import pallas as pl
from jax.experimental.pallas import tpu as pltpu
from jax.experimental.pallas import tpu_sc as plsc

_VOCAB_N = 100000
_D = 300
_B = 1024
_CTX = 20

_NC = 2
_NS = 16
_NW = _NC * _NS

_ROWS = _B * _CTX
_ROWS_PER_W = _ROWS // _NW
_CH = 128
_NCHUNK = _ROWS_PER_W // _CH


_DP = 384


def _sc_gather(table, idx_flat):
    mesh = plsc.VectorSubcoreMesh(core_axis_name="c", subcore_axis_name="s")

    @functools.partial(
        pl.kernel,
        mesh=mesh,
        out_type=jax.ShapeDtypeStruct((_ROWS, _DP), jnp.float32),
        scratch_types=[
            pltpu.VMEM((_CH,), jnp.int32),
            pltpu.VMEM((_CH, 256), jnp.float32),
            pltpu.VMEM((_CH, 128), jnp.float32),
            pltpu.SemaphoreType.DMA,
        ],
    )
    def k(table_hbm, idx_hbm, rows_hbm, idx_v, r01, r2, sem):
        wid = lax.axis_index("s") * _NC + lax.axis_index("c")
        base = wid * _ROWS_PER_W

        def chunk(t, carry):
            off = base + t * _CH
            pltpu.sync_copy(idx_hbm.at[pl.ds(off, _CH)], idx_v)
            pltpu.async_copy(table_hbm.at[idx_v, pl.ds(0, 256)], r01, sem).wait()
            o2 = pl.multiple_of(t * 0 + 256, 128)
            pltpu.async_copy(table_hbm.at[idx_v, pl.ds(o2, 128)], r2, sem).wait()
            pltpu.sync_copy(r01, rows_hbm.at[pl.ds(off, _CH), pl.ds(0, 256)])
            pltpu.sync_copy(r2, rows_hbm.at[pl.ds(off, _CH), pl.ds(256, 128)])
            return carry

        lax.fori_loop(0, _NCHUNK, chunk, 0)

    return k(table, idx_flat)


def _pool_body(e_ref, o_ref):
    ep = e_ref[...]
    e = ep[..., :_D]
    sq = jnp.sum(e * e, axis=-1, keepdims=True)
    norm = jnp.sqrt(sq)
    scale = jnp.minimum(1.0, 1.0 / (norm + 1e-7))
    o_ref[...] = jnp.mean(e * scale, axis=1)


def _pool(rows):
    bb = 128
    e = rows.reshape(_B, _CTX, _DP)
    return pl.pallas_call(
        _pool_body,
        grid=(_B // bb,),
        in_specs=[pl.BlockSpec((bb, _CTX, _DP), lambda i: (i, 0, 0))],
        out_specs=pl.BlockSpec((bb, _D), lambda i: (i, 0)),
        out_shape=jax.ShapeDtypeStruct((_B, _D), jnp.float32),
    )(e)


_MB = 2048
_NFULL = _VOCAB_N // _MB
_EP_BASE = _NFULL * _MB
_EP_N = _VOCAB_N - _EP_BASE
_EP_PAD = 100096 - _EP_BASE


def _mm2_body(x_ref, b_ref, w_hbm, o_ref, xb_ref, wbuf, obuf, wsem, osem):
    xb_ref[...] = x_ref[...].astype(jnp.bfloat16)
    t0 = (b_ref[0, 0] * 0.0).astype(jnp.int32)

    def wcp(row_off, slot):
        return pltpu.make_async_copy(
            w_hbm.at[pl.ds(row_off, _MB)], wbuf.at[slot], wsem.at[slot]
        )

    def ocp(col_off, slot, size):
        off = pl.multiple_of(t0 + col_off, 128)
        return pltpu.make_async_copy(
            obuf.at[slot, slice(None), pl.ds(0, size)],
            o_ref.at[:, pl.ds(off, size)],
            osem.at[slot],
        )

    def compute(slot, col_off, nsize):
        wb = wbuf[slot, pl.ds(0, nsize), :].astype(jnp.bfloat16)
        acc = lax.dot_general(
            xb_ref[...], wb, (((1,), (1,)), ((), ())),
            preferred_element_type=jnp.float32,
        )
        boff = pl.multiple_of(t0 + col_off, 128)
        bv = b_ref[0, pl.ds(boff, nsize)]
        obuf[slot, slice(None), pl.ds(0, nsize)] = acc + bv[None, :]

    wcp(0, 0).start()
    wcp(_MB, 1).start()
    wcp(2 * _MB, 2).start()

    def step(j, carry):
        slot = lax.rem(j, 3)
        wcp(j * _MB, slot).wait()

        @pl.when(j >= 3)
        def _():
            ocp(j * _MB, slot, _MB).wait()

        compute(slot, j * _MB, _MB)
        ocp(j * _MB, slot, _MB).start()

        @pl.when(j + 3 < _NFULL)
        def _():
            wcp((j + 3) * _MB, slot).start()

        return carry

    lax.fori_loop(0, _NFULL, step, 0)

    ep_w = pltpu.make_async_copy(
        w_hbm.at[pl.ds(_EP_BASE, _EP_N)],
        wbuf.at[0, pl.ds(0, _EP_N)],
        wsem.at[0],
    )
    ep_w.start()
    ocp((_NFULL - 3) * _MB, 0, _MB).wait()
    ep_w.wait()
    compute(0, _EP_BASE, _EP_PAD)
    ocp(_EP_BASE, 0, _EP_PAD).start()
    ocp((_NFULL - 2) * _MB, 1, _MB).wait()
    ocp((_NFULL - 1) * _MB, 2, _MB).wait()
    ocp(_EP_BASE, 0, _EP_PAD).wait()


def _project(x, W, b):
    b2 = b.reshape(1, _VOCAB_N)
    return pl.pallas_call(
        _mm2_body,
        in_specs=[
            pl.BlockSpec((_B, _D), lambda: (0, 0)),
            pl.BlockSpec((1, _VOCAB_N), lambda: (0, 0)),
            pl.BlockSpec(memory_space=pl.ANY),
        ],
        out_specs=pl.BlockSpec(memory_space=pl.ANY),
        out_shape=jax.ShapeDtypeStruct((_B, _VOCAB_N), jnp.float32),
        scratch_shapes=[
            pltpu.VMEM((_B, _D), jnp.bfloat16),
            pltpu.VMEM((3, _MB, _D), jnp.float32),
            pltpu.VMEM((3, _B, _MB), jnp.float32),
            pltpu.SemaphoreType.DMA((3,)),
            pltpu.SemaphoreType.DMA((3,)),
        ],
    )(x, b2, W)


def kernel(inputs_, emb_table, W, b):
    idx_flat = inputs_.reshape(-1).astype(jnp.int32)
    rows = _sc_gather(emb_table, idx_flat)
    x = _pool(rows)
    return _project(x, W, b)


# speedup vs baseline: 1.0817x; 1.0787x over previous
import functools

import jax
import jax.numpy as jnp
from jax import lax
from jax.experimental import pallas as pl
from jax.experimental.pallas import tpu as pltpu
from jax.experimental.pallas import tpu_sc as plsc

_VOCAB_N = 100000
_D = 300
_B = 1024
_CTX = 20

_NC = 2
_NS = 16
_NW = _NC * _NS

_ROWS = _B * _CTX
_ROWS_PER_W = _ROWS // _NW
_CH = 128
_NCHUNK = _ROWS_PER_W // _CH


_DP = 384


def _sc_gather_pool(table, idx_flat):
    mesh = plsc.VectorSubcoreMesh(core_axis_name="c", subcore_axis_name="s")
    ew = 32
    ec = 8
    nch = ew // ec
    rows_c = ec * _CTX

    @functools.partial(
        pl.kernel,
        mesh=mesh,
        compiler_params=pltpu.CompilerParams(needs_layout_passes=False),
        out_type=jax.ShapeDtypeStruct((_B, 304), jnp.float32),
        scratch_types=[
            pltpu.VMEM((2, 80), jnp.int32),
            pltpu.VMEM((rows_c, 256), jnp.float32),
            pltpu.VMEM((rows_c, 128), jnp.float32),
            pltpu.VMEM((ec, 304), jnp.float32),
            pltpu.SMEM((32,), jnp.float32),
            pltpu.SemaphoreType.DMA,
        ],
    )
    def k(table_hbm, idx_hbm, x_hbm, idx_v, r01, r2, xv, sc_ref, sem):
        wid = lax.axis_index("s") * _NC + lax.axis_index("c")
        lanes = lax.iota(jnp.int32, 16)
        tail_ok = lanes < 12
        magic = jnp.full((16,), 0x5F3759DF, dtype=jnp.int32)

        def rsq_slices(r):
            out = []
            for kk in range(16):
                out.append((r01, r, 16 * kk, None))
            out.append((r2, r, 0, None))
            out.append((r2, r, 16, None))
            out.append((r2, r, 32, tail_ok))
            return out

        def load_slice(buf, r, off, msk):
            v = buf[r, pl.ds(off, 16)]
            if msk is not None:
                v = jnp.where(msk, v, 0.0)
            return v

        def chunk(t, carry):
            base_e = wid * ew + t * ec
            roff = base_e * _CTX
            pltpu.sync_copy(idx_hbm.at[pl.ds(roff, 80)], idx_v.at[0])
            pltpu.sync_copy(idx_hbm.at[pl.ds(roff + 80, 80)], idx_v.at[1])
            o2 = pl.multiple_of(t * 0 + 256, 128)
            for h in range(2):
                pltpu.async_copy(
                    table_hbm.at[idx_v.at[h], pl.ds(0, 256)],
                    r01.at[pl.ds(80 * h, 80)], sem).wait()
                pltpu.async_copy(
                    table_hbm.at[idx_v.at[h], pl.ds(o2, 128)],
                    r2.at[pl.ds(80 * h, 80)], sem).wait()

            def elem(e, c2):
                def p1(c, c3):
                    r = e * _CTX + c
                    acc = jnp.zeros((16,), jnp.float32)
                    for sl in rsq_slices(r):
                        v = load_slice(*sl)
                        acc = acc + v * v
                    sca = jnp.full((16,), jnp.sum(acc))
                    i = plsc.bitcast(sca, jnp.int32)
                    y = plsc.bitcast(
                        magic - lax.shift_right_logical(i, 1), jnp.float32)
                    for _ in range(3):
                        y = y * (1.5 - 0.5 * sca * y * y)
                    norm = sca * y
                    scale = jnp.minimum(1.0, 1.0 / (norm + 1e-7))
                    sc_ref[c] = jnp.max(scale)
                    return c3

                lax.fori_loop(0, _CTX, p1, 0)

                def p2(c, accs):
                    r = e * _CTX + c
                    g = sc_ref[c]
                    return tuple(
                        a + g * load_slice(*sl)
                        for a, sl in zip(accs, rsq_slices(r))
                    )

                accs = lax.fori_loop(
                    0, _CTX, p2,
                    tuple(jnp.zeros((16,), jnp.float32) for _ in range(19)))
                inv = 1.0 / _CTX
                for kk in range(19):
                    xv[e, pl.ds(16 * kk, 16)] = accs[kk] * inv
                return c2

            lax.fori_loop(0, ec, elem, 0)
            pltpu.sync_copy(xv, x_hbm.at[pl.ds(base_e, ec)])
            return carry

        lax.fori_loop(0, nch, chunk, 0)

    return k(table, idx_flat)


_MB = 2048
_NFULL = _VOCAB_N // _MB
_EP_BASE = _NFULL * _MB
_EP_N = _VOCAB_N - _EP_BASE
_EP_PAD = 100096 - _EP_BASE


def _mm2_body(x_ref, b_ref, w_hbm, o_ref, xb_ref, wbuf, obuf, wsem, osem):
    xb_ref[...] = x_ref[...][:, :_D].astype(jnp.bfloat16)
    t0 = (b_ref[0, 0] * 0.0).astype(jnp.int32)

    def wcp(row_off, slot):
        return pltpu.make_async_copy(
            w_hbm.at[pl.ds(row_off, _MB)], wbuf.at[slot], wsem.at[slot]
        )

    def ocp(col_off, slot, size):
        off = pl.multiple_of(t0 + col_off, 128)
        return pltpu.make_async_copy(
            obuf.at[slot, slice(None), pl.ds(0, size)],
            o_ref.at[:, pl.ds(off, size)],
            osem.at[slot],
        )

    def compute(slot, col_off, nsize):
        wb = wbuf[slot, pl.ds(0, nsize), :].astype(jnp.bfloat16)
        acc = lax.dot_general(
            xb_ref[...], wb, (((1,), (1,)), ((), ())),
            preferred_element_type=jnp.float32,
        )
        boff = pl.multiple_of(t0 + col_off, 128)
        bv = b_ref[0, pl.ds(boff, nsize)]
        obuf[slot, slice(None), pl.ds(0, nsize)] = acc + bv[None, :]

    wcp(0, 0).start()
    wcp(_MB, 1).start()
    wcp(2 * _MB, 2).start()

    def step(j, carry):
        slot = lax.rem(j, 3)
        wcp(j * _MB, slot).wait()

        @pl.when(j >= 3)
        def _():
            ocp(j * _MB, slot, _MB).wait()

        compute(slot, j * _MB, _MB)
        ocp(j * _MB, slot, _MB).start()

        @pl.when(j + 3 < _NFULL)
        def _():
            wcp((j + 3) * _MB, slot).start()

        return carry

    lax.fori_loop(0, _NFULL, step, 0)

    ep_w = pltpu.make_async_copy(
        w_hbm.at[pl.ds(_EP_BASE, _EP_N)],
        wbuf.at[0, pl.ds(0, _EP_N)],
        wsem.at[0],
    )
    ep_w.start()
    ocp((_NFULL - 3) * _MB, 0, _MB).wait()
    ep_w.wait()
    compute(0, _EP_BASE, _EP_PAD)
    ocp(_EP_BASE, 0, _EP_PAD).start()
    ocp((_NFULL - 2) * _MB, 1, _MB).wait()
    ocp((_NFULL - 1) * _MB, 2, _MB).wait()
    ocp(_EP_BASE, 0, _EP_PAD).wait()


def _project(x, W, b):
    b2 = b.reshape(1, _VOCAB_N)
    return pl.pallas_call(
        _mm2_body,
        in_specs=[
            pl.BlockSpec((_B, 304), lambda: (0, 0)),
            pl.BlockSpec((1, _VOCAB_N), lambda: (0, 0)),
            pl.BlockSpec(memory_space=pl.ANY),
        ],
        out_specs=pl.BlockSpec(memory_space=pl.ANY),
        out_shape=jax.ShapeDtypeStruct((_B, _VOCAB_N), jnp.float32),
        scratch_shapes=[
            pltpu.VMEM((_B, _D), jnp.bfloat16),
            pltpu.VMEM((3, _MB, _D), jnp.float32),
            pltpu.VMEM((3, _B, _MB), jnp.float32),
            pltpu.SemaphoreType.DMA((3,)),
            pltpu.SemaphoreType.DMA((3,)),
        ],
    )(x, b2, W)


def kernel(inputs_, emb_table, W, b):
    idx_flat = inputs_.reshape(-1).astype(jnp.int32)
    x = _sc_gather_pool(emb_table, idx_flat)
    return _project(x, W, b)
